# Initial kernel scaffold; baseline (speedup 1.0000x reference)
import jax
import jax.numpy as jnp
from jax.experimental import pallas as pl


def kernel(input_tensor, gene_W1, gene_b1, gene_W2, gene_b2, go_enc_W, go_enc_b, go_dec_W1, go_dec_b1, go_dec_W2, go_dec_b2, ke_ws0, ke_wn0, ke_b0, ke_ws1, ke_wn1, ke_b1, kel_W1, kel_b1, kel_W2, kel_b2, bio_W1, bio_b1, bio_W2, bio_b2, drug_W1, drug_b1, drug_W2, drug_b2, pred_W, pred_b, gene_go, go_ke, ke_ke, tissue):
    raise NotImplementedError("write your pallas kernel here")


# speedup vs baseline: 1.1108x; 1.1108x over previous
import functools

import jax
import jax.numpy as jnp
from jax import lax
from jax.experimental import pallas as pl
from jax.experimental.pallas import tpu as pltpu
from jax.experimental.pallas import tpu_sc as plsc

B = 256
NG = 8192
NGO = 4096
NKE = 1024
DRUG = 2048
NT = 512
H = 8
NC = 2
NS = 16
NW = NC * NS
C = 128

_F32 = jnp.float32
_HIGH = lax.Precision.HIGHEST


def _dotT(w, x):
    return lax.dot_general(w, x, (((0,), (0,)), ((), ())),
                           preferred_element_type=_F32, precision=_HIGH)


@functools.lru_cache(maxsize=None)
def _make_edge_mean(nsrc, ndst, ne):
    epw = ne // NW
    nch = epw // C
    rps = ndst // NS
    mesh = plsc.VectorSubcoreMesh(core_axis_name="c", subcore_axis_name="s", num_cores=NC, num_subcores=NS)

    @functools.partial(
        pl.kernel,
        out_type=(jax.ShapeDtypeStruct((NC, ndst, B), _F32),
                  jax.ShapeDtypeStruct((NC, ndst, B), _F32)),
        mesh=mesh,
        scratch_types=[
            pltpu.VMEM((nch, C), jnp.int32),
            pltpu.VMEM((nch, C), jnp.int32),
            pltpu.VMEM((C, B), _F32),
            pltpu.VMEM((C, B), _F32),
            pltpu.SemaphoreType.DMA,
        ],
    )
    def ker(x_hbm, src_hbm, dst_hbm, zeros_hbm, ones_hbm,
            sums_hbm, cnts_hbm, srcv, dstv, rows, ones, sem):
        cid = lax.axis_index("c")
        sid = lax.axis_index("s")
        w = cid * NS + sid
        pltpu.sync_copy(zeros_hbm.at[pl.ds(0, rps)],
                        sums_hbm.at[cid, pl.ds(sid * rps, rps)])
        pltpu.sync_copy(zeros_hbm.at[pl.ds(0, rps)],
                        cnts_hbm.at[cid, pl.ds(sid * rps, rps)])
        pltpu.sync_copy(src_hbm.at[w], srcv)
        pltpu.sync_copy(dst_hbm.at[w], dstv)
        pltpu.sync_copy(ones_hbm, ones)
        plsc.subcore_barrier()

        def body(j, carry):
            pltpu.async_copy(x_hbm.at[srcv.at[j]], rows, sem).wait()
            pltpu.sync_copy(rows, sums_hbm.at[cid].at[dstv.at[j]], add=True)
            pltpu.sync_copy(ones, cnts_hbm.at[cid].at[dstv.at[j]], add=True)
            return carry

        lax.fori_loop(0, nch, body, 0)

    return ker


@functools.lru_cache(maxsize=None)
def _make_row_gather(nsrc, nidx):
    ipw = nidx // NW
    mesh = plsc.VectorSubcoreMesh(core_axis_name="c", subcore_axis_name="s", num_cores=NC, num_subcores=NS)

    @functools.partial(
        pl.kernel,
        out_type=jax.ShapeDtypeStruct((nidx, B), _F32),
        mesh=mesh,
        scratch_types=[
            pltpu.VMEM((ipw,), jnp.int32),
            pltpu.VMEM((ipw, B), _F32),
            pltpu.SemaphoreType.DMA,
        ],
    )
    def ker(x_hbm, idx_hbm, out_hbm, idxv, rows, sem):
        w = lax.axis_index("c") * NS + lax.axis_index("s")
        pltpu.sync_copy(idx_hbm.at[pl.ds(w * ipw, ipw)], idxv)
        pltpu.async_copy(x_hbm.at[idxv], rows, sem).wait()
        pltpu.sync_copy(rows, out_hbm.at[pl.ds(w * ipw, ipw)])

    return ker


def _node_mlp_body(x, w1_ref, b1_ref, w2_ref, b2_ref):
    acc = jnp.zeros_like(x)
    for k in range(H):
        acc = acc + jnp.tanh(x * w1_ref[k] + b1_ref[k]) * w2_ref[k]
    return acc + b2_ref[0]


def _tc_node_mlp(x_t, w1, b1, w2, b2, bn):
    n = x_t.shape[0]

    def body(x_ref, w1_ref, b1_ref, w2_ref, b2_ref, o_ref):
        o_ref[...] = _node_mlp_body(x_ref[...], w1_ref, b1_ref, w2_ref, b2_ref)

    smem = pl.BlockSpec(memory_space=pltpu.SMEM)
    return pl.pallas_call(
        body,
        grid=(n // bn,),
        in_specs=[pl.BlockSpec((bn, B), lambda j: (j, 0)), smem, smem, smem, smem],
        out_specs=pl.BlockSpec((bn, B), lambda j: (j, 0)),
        out_shape=jax.ShapeDtypeStruct((n, B), _F32),
    )(x_t, w1, b1, w2, b2)


def _combine_mean(s_ref, c_ref):
    s = s_ref[0] + s_ref[1]
    c = c_ref[0, :, 0:1] + c_ref[1, :, 0:1]
    return s / jnp.maximum(c, 1.0)


def _tc_mean(sums, cnts):
    ndst = sums.shape[1]

    def body(s_ref, c_ref, o_ref):
        o_ref[...] = _combine_mean(s_ref, c_ref)

    return pl.pallas_call(
        body,
        out_shape=jax.ShapeDtypeStruct((ndst, B), _F32),
    )(sums, cnts)


def _tc_go_encode(mean, enc_w, enc_b2d, dw1, db1, dw2, db2):
    OB = 512

    def body(m_ref, w_ref, b_ref, w1_ref, b1_ref, w2_ref, b2_ref, o_ref):
        g = jnp.tanh(_dotT(w_ref[...], m_ref[...]) + b_ref[...])
        o_ref[...] = _node_mlp_body(g, w1_ref, b1_ref, w2_ref, b2_ref)

    smem = pl.BlockSpec(memory_space=pltpu.SMEM)
    return pl.pallas_call(
        body,
        grid=(NGO // OB,),
        in_specs=[
            pl.BlockSpec((NGO, B), lambda j: (0, 0)),
            pl.BlockSpec((NGO, OB), lambda j: (0, j)),
            pl.BlockSpec((OB, 1), lambda j: (j, 0)),
            smem, smem, smem, smem,
        ],
        out_specs=pl.BlockSpec((OB, B), lambda j: (j, 0)),
        out_shape=jax.ShapeDtypeStruct((NGO, B), _F32),
    )(mean, enc_w, enc_b2d, dw1, db1, dw2, db2)


def _tc_ke_mix(sums, cnts, ke, ws2d, wn2d, b2d):
    def body(s_ref, c_ref, k_ref, ws_ref, wn_ref, b_ref, o_ref):
        agg = _combine_mean(s_ref, c_ref)
        o_ref[...] = jax.nn.relu(k_ref[...] * ws_ref[...] + agg * wn_ref[...]
                                 + b_ref[...])

    return pl.pallas_call(
        body,
        out_shape=jax.ShapeDtypeStruct((NKE, B), _F32),
    )(sums, cnts, ke, ws2d, wn2d, b2d)


def _tc_ke_mix_mlp(sums, cnts, ke, ws2d, wn2d, b2d, w1, b1, w2, b2):
    def body(s_ref, c_ref, k_ref, ws_ref, wn_ref, b_ref,
             w1_ref, b1_ref, w2_ref, b2_ref, o_ref):
        agg = _combine_mean(s_ref, c_ref)
        mixed = jax.nn.relu(k_ref[...] * ws_ref[...] + agg * wn_ref[...]
                            + b_ref[...])
        o_ref[...] = _node_mlp_body(mixed, w1_ref, b1_ref, w2_ref, b2_ref)

    smem = pl.BlockSpec(memory_space=pltpu.SMEM)
    vmem = pl.BlockSpec()
    return pl.pallas_call(
        body,
        in_specs=[vmem, vmem, vmem, vmem, vmem, vmem, smem, smem, smem, smem],
        out_shape=jax.ShapeDtypeStruct((NKE, B), _F32),
    )(sums, cnts, ke, ws2d, wn2d, b2d, w1, b1, w2, b2)


def _tc_heads(bio_t, drug_t, bw1, bb1, bw2, bb2, dw1, db1, dw2, db2, pw, pb):
    def body(bio_ref, drug_ref, bw1_ref, bb1_ref, bw2_ref, bb2_ref,
             dw1_ref, db1_ref, dw2_ref, db2_ref, pw_ref, pb_ref, o_ref):
        h = jax.nn.relu(_dotT(bw1_ref[...], bio_ref[...]) + bb1_ref[...])
        h = jax.nn.relu(_dotT(bw2_ref[...], h) + bb2_ref[...])
        d = jax.nn.relu(_dotT(dw1_ref[...], drug_ref[...]) + db1_ref[...])
        d = jax.nn.relu(_dotT(dw2_ref[...], d) + db2_ref[...])
        comb = jnp.concatenate([h, d], axis=0)
        o_ref[...] = _dotT(pw_ref[...], comb) + pb_ref[...]

    return pl.pallas_call(
        body,
        out_shape=jax.ShapeDtypeStruct((1, B), _F32),
    )(bio_t, drug_t, bw1, bb1, bw2, bb2, dw1, db1, dw2, db2, pw, pb)


def kernel(input_tensor, gene_W1, gene_b1, gene_W2, gene_b2, go_enc_W,
           go_enc_b, go_dec_W1, go_dec_b1, go_dec_W2, go_dec_b2,
           ke_ws0, ke_wn0, ke_b0, ke_ws1, ke_wn1, ke_b1,
           kel_W1, kel_b1, kel_W2, kel_b2,
           bio_W1, bio_b1, bio_W2, bio_b2,
           drug_W1, drug_b1, drug_W2, drug_b2,
           pred_W, pred_b,
           gene_go, go_ke, ke_ke, tissue):
    e_gg = gene_go.shape[1]
    e_gk = go_ke.shape[1]
    e_kk = ke_ke.shape[1]

    input_t = input_tensor.T
    xg_t = input_t[:NG]
    drug_t = input_t[NG:]

    zeros = jnp.zeros((256, B), _F32)
    ones = jnp.ones((C, B), _F32)

    def chunked(idx, ne):
        return idx.reshape(NW, ne // NW // C, C)

    src_gg, dst_gg = chunked(gene_go[0], e_gg), chunked(gene_go[1], e_gg)
    src_gk, dst_gk = chunked(go_ke[0], e_gk), chunked(go_ke[1], e_gk)
    src_kk, dst_kk = chunked(ke_ke[0], e_kk), chunked(ke_ke[1], e_kk)

    col = lambda v: v.reshape(-1, 1)

    gene_t = _tc_node_mlp(xg_t, gene_W1, gene_b1, gene_W2, gene_b2, 1024)
    s1, c1 = _make_edge_mean(NG, NGO, e_gg)(gene_t, src_gg, dst_gg, zeros, ones)
    mean1 = _tc_mean(s1, c1)
    godec_t = _tc_go_encode(mean1, go_enc_W, col(go_enc_b),
                            go_dec_W1, go_dec_b1, go_dec_W2, go_dec_b2)

    s2, c2 = _make_edge_mean(NGO, NKE, e_gk)(godec_t, src_gk, dst_gk, zeros, ones)
    ke0 = _tc_mean(s2, c2)

    s3, c3 = _make_edge_mean(NKE, NKE, e_kk)(ke0, src_kk, dst_kk, zeros, ones)
    ke1 = _tc_ke_mix(s3, c3, ke0, col(ke_ws0), col(ke_wn0), col(ke_b0))
    s4, c4 = _make_edge_mean(NKE, NKE, e_kk)(ke1, src_kk, dst_kk, zeros, ones)
    ke2 = _tc_ke_mix_mlp(s4, c4, ke1, col(ke_ws1), col(ke_wn1), col(ke_b1),
                         kel_W1, kel_b1, kel_W2, kel_b2)

    bio_t = _make_row_gather(NKE, NT)(ke2, tissue)
    out_t = _tc_heads(bio_t, drug_t, bio_W1, col(bio_b1), bio_W2, col(bio_b2),
                      drug_W1, col(drug_b1), drug_W2, col(drug_b2),
                      pred_W, col(pred_b))
    return out_t.reshape(B, 1)


# speedup vs baseline: 1.2233x; 1.1012x over previous
import functools

import jax
import jax.numpy as jnp
from jax import lax
from jax.experimental import pallas as pl
from jax.experimental.pallas import tpu as pltpu
from jax.experimental.pallas import tpu_sc as plsc

B = 256
NG = 8192
NGO = 4096
NKE = 1024
DRUG = 2048
NT = 512
H = 8
NC = 2
NS = 16
NW = NC * NS
C = 128

_F32 = jnp.float32
_HIGH = lax.Precision.HIGHEST


def _dotT(w, x):
    return lax.dot_general(w, x, (((0,), (0,)), ((), ())),
                           preferred_element_type=_F32, precision=_HIGH)


@functools.lru_cache(maxsize=None)
def _make_edge_mean(nsrc, ndst, ne, with_counts=True):
    epw = ne // NW
    nch = epw // C
    rps = ndst // NS
    mesh = plsc.VectorSubcoreMesh(core_axis_name="c", subcore_axis_name="s", num_cores=NC, num_subcores=NS)

    out_type = [jax.ShapeDtypeStruct((NC, ndst, B), _F32)]
    if with_counts:
        out_type.append(jax.ShapeDtypeStruct((NC, ndst, B), _F32))

    @functools.partial(
        pl.kernel,
        out_type=tuple(out_type) if with_counts else out_type[0],
        mesh=mesh,
        scratch_types=[
            pltpu.VMEM((nch, C), jnp.int32),
            pltpu.VMEM((nch, C), jnp.int32),
            pltpu.VMEM((C, B), _F32),
            pltpu.VMEM((C, B), _F32),
            pltpu.VMEM((C, B), _F32),
            pltpu.SemaphoreType.DMA,
            pltpu.SemaphoreType.DMA,
        ],
    )
    def ker(x_hbm, src_hbm, dst_hbm, zeros_hbm, ones_hbm,
            sums_hbm, *rest):
        if with_counts:
            cnts_hbm, srcv, dstv, rows0, rows1, ones, sem0, sem1 = rest
        else:
            srcv, dstv, rows0, rows1, ones, sem0, sem1 = rest
            cnts_hbm = None
        cid = lax.axis_index("c")
        sid = lax.axis_index("s")
        w = cid * NS + sid
        pltpu.sync_copy(zeros_hbm.at[pl.ds(0, rps)],
                        sums_hbm.at[cid, pl.ds(sid * rps, rps)])
        if with_counts:
            pltpu.sync_copy(zeros_hbm.at[pl.ds(0, rps)],
                            cnts_hbm.at[cid, pl.ds(sid * rps, rps)])
        pltpu.sync_copy(src_hbm.at[w], srcv)
        pltpu.sync_copy(dst_hbm.at[w], dstv)
        pltpu.sync_copy(ones_hbm, ones)
        plsc.subcore_barrier()

        def scatter(j, rows):
            pltpu.sync_copy(rows, sums_hbm.at[cid].at[dstv.at[j]], add=True)
            if with_counts:
                pltpu.sync_copy(ones, cnts_hbm.at[cid].at[dstv.at[j]], add=True)

        pltpu.async_copy(x_hbm.at[srcv.at[0]], rows0, sem0)

        def pair(i, carry):
            j = i * 2
            pltpu.make_async_copy(x_hbm.at[srcv.at[j]], rows0, sem0).wait()
            pltpu.async_copy(x_hbm.at[srcv.at[j + 1]], rows1, sem1)
            scatter(j, rows0)
            pltpu.make_async_copy(x_hbm.at[srcv.at[j + 1]], rows1, sem1).wait()

            @pl.when(i + 1 < nch // 2)
            def _():
                pltpu.async_copy(x_hbm.at[srcv.at[j + 2]], rows0, sem0)

            scatter(j + 1, rows1)
            return carry

        lax.fori_loop(0, nch // 2, pair, 0)

    return ker


@functools.lru_cache(maxsize=None)
def _make_row_gather(nsrc, nidx):
    ipw = nidx // NW
    mesh = plsc.VectorSubcoreMesh(core_axis_name="c", subcore_axis_name="s", num_cores=NC, num_subcores=NS)

    @functools.partial(
        pl.kernel,
        out_type=jax.ShapeDtypeStruct((nidx, B), _F32),
        mesh=mesh,
        scratch_types=[
            pltpu.VMEM((ipw,), jnp.int32),
            pltpu.VMEM((ipw, B), _F32),
            pltpu.SemaphoreType.DMA,
        ],
    )
    def ker(x_hbm, idx_hbm, out_hbm, idxv, rows, sem):
        w = lax.axis_index("c") * NS + lax.axis_index("s")
        pltpu.sync_copy(idx_hbm.at[pl.ds(w * ipw, ipw)], idxv)
        pltpu.async_copy(x_hbm.at[idxv], rows, sem).wait()
        pltpu.sync_copy(rows, out_hbm.at[pl.ds(w * ipw, ipw)])

    return ker


def _node_mlp_body(x, w1_ref, b1_ref, w2_ref, b2_ref):
    acc = jnp.zeros_like(x)
    for k in range(H):
        acc = acc + jnp.tanh(x * w1_ref[k] + b1_ref[k]) * w2_ref[k]
    return acc + b2_ref[0]


def _tc_node_mlp(x_t, w1, b1, w2, b2, bn):
    n = x_t.shape[0]

    def body(x_ref, w1_ref, b1_ref, w2_ref, b2_ref, o_ref):
        o_ref[...] = _node_mlp_body(x_ref[...], w1_ref, b1_ref, w2_ref, b2_ref)

    smem = pl.BlockSpec(memory_space=pltpu.SMEM)
    return pl.pallas_call(
        body,
        grid=(n // bn,),
        in_specs=[pl.BlockSpec((bn, B), lambda j: (j, 0)), smem, smem, smem, smem],
        out_specs=pl.BlockSpec((bn, B), lambda j: (j, 0)),
        out_shape=jax.ShapeDtypeStruct((n, B), _F32),
    )(x_t, w1, b1, w2, b2)


def _combine_mean(s_ref, c_ref):
    s = s_ref[0] + s_ref[1]
    c = c_ref[0, :, 0:1] + c_ref[1, :, 0:1]
    return s / jnp.maximum(c, 1.0)


def _tc_mean(sums, cnts):
    ndst = sums.shape[1]

    def body(s_ref, c_ref, o_ref):
        o_ref[...] = _combine_mean(s_ref, c_ref)

    return pl.pallas_call(
        body,
        out_shape=jax.ShapeDtypeStruct((ndst, B), _F32),
    )(sums, cnts)


def _tc_go_encode(sums, cnts, enc_w, enc_b2d, dw1, db1, dw2, db2):
    OB = 512

    def body(s_ref, c_ref, w_ref, b_ref, w1_ref, b1_ref, w2_ref, b2_ref,
             o_ref, m_ref):
        @pl.when(pl.program_id(0) == 0)
        def _():
            m_ref[...] = _combine_mean(s_ref, c_ref)

        g = jnp.tanh(_dotT(w_ref[...], m_ref[...]) + b_ref[...])
        o_ref[...] = _node_mlp_body(g, w1_ref, b1_ref, w2_ref, b2_ref)

    smem = pl.BlockSpec(memory_space=pltpu.SMEM)
    return pl.pallas_call(
        body,
        grid=(NGO // OB,),
        in_specs=[
            pl.BlockSpec((NC, NGO, B), lambda j: (0, 0, 0)),
            pl.BlockSpec((NC, NGO, B), lambda j: (0, 0, 0)),
            pl.BlockSpec((NGO, OB), lambda j: (0, j)),
            pl.BlockSpec((OB, 1), lambda j: (j, 0)),
            smem, smem, smem, smem,
        ],
        out_specs=pl.BlockSpec((OB, B), lambda j: (j, 0)),
        out_shape=jax.ShapeDtypeStruct((NGO, B), _F32),
        scratch_shapes=[pltpu.VMEM((NGO, B), _F32)],
    )(sums, cnts, enc_w, enc_b2d, dw1, db1, dw2, db2)


def _tc_ke_mix(sums, cnts, ke, ws2d, wn2d, b2d):
    def body(s_ref, c_ref, k_ref, ws_ref, wn_ref, b_ref, o_ref):
        agg = _combine_mean(s_ref, c_ref)
        o_ref[...] = jax.nn.relu(k_ref[...] * ws_ref[...] + agg * wn_ref[...]
                                 + b_ref[...])

    return pl.pallas_call(
        body,
        out_shape=jax.ShapeDtypeStruct((NKE, B), _F32),
    )(sums, cnts, ke, ws2d, wn2d, b2d)


def _tc_ke_mix_mlp(sums, cnts, ke, ws2d, wn2d, b2d, w1, b1, w2, b2):
    def body(s_ref, c_ref, k_ref, ws_ref, wn_ref, b_ref,
             w1_ref, b1_ref, w2_ref, b2_ref, o_ref):
        agg = _combine_mean(s_ref, c_ref)
        mixed = jax.nn.relu(k_ref[...] * ws_ref[...] + agg * wn_ref[...]
                            + b_ref[...])
        o_ref[...] = _node_mlp_body(mixed, w1_ref, b1_ref, w2_ref, b2_ref)

    smem = pl.BlockSpec(memory_space=pltpu.SMEM)
    vmem = pl.BlockSpec()
    return pl.pallas_call(
        body,
        in_specs=[vmem, vmem, vmem, vmem, vmem, vmem, smem, smem, smem, smem],
        out_shape=jax.ShapeDtypeStruct((NKE, B), _F32),
    )(sums, cnts, ke, ws2d, wn2d, b2d, w1, b1, w2, b2)


def _tc_heads(bio_t, drug_t, bw1, bb1, bw2, bb2, dw1, db1, dw2, db2, pw, pb):
    def body(bio_ref, drug_ref, bw1_ref, bb1_ref, bw2_ref, bb2_ref,
             dw1_ref, db1_ref, dw2_ref, db2_ref, pw_ref, pb_ref, o_ref):
        h = jax.nn.relu(_dotT(bw1_ref[...], bio_ref[...]) + bb1_ref[...])
        h = jax.nn.relu(_dotT(bw2_ref[...], h) + bb2_ref[...])
        d = jax.nn.relu(_dotT(dw1_ref[...], drug_ref[...]) + db1_ref[...])
        d = jax.nn.relu(_dotT(dw2_ref[...], d) + db2_ref[...])
        comb = jnp.concatenate([h, d], axis=0)
        o_ref[...] = _dotT(pw_ref[...], comb) + pb_ref[...]

    return pl.pallas_call(
        body,
        out_shape=jax.ShapeDtypeStruct((1, B), _F32),
    )(bio_t, drug_t, bw1, bb1, bw2, bb2, dw1, db1, dw2, db2, pw, pb)


def kernel(input_tensor, gene_W1, gene_b1, gene_W2, gene_b2, go_enc_W,
           go_enc_b, go_dec_W1, go_dec_b1, go_dec_W2, go_dec_b2,
           ke_ws0, ke_wn0, ke_b0, ke_ws1, ke_wn1, ke_b1,
           kel_W1, kel_b1, kel_W2, kel_b2,
           bio_W1, bio_b1, bio_W2, bio_b2,
           drug_W1, drug_b1, drug_W2, drug_b2,
           pred_W, pred_b,
           gene_go, go_ke, ke_ke, tissue):
    e_gg = gene_go.shape[1]
    e_gk = go_ke.shape[1]
    e_kk = ke_ke.shape[1]

    input_t = input_tensor.T
    xg_t = input_t[:NG]
    drug_t = input_t[NG:]

    zeros = jnp.zeros((256, B), _F32)
    ones = jnp.ones((C, B), _F32)

    def chunked(idx, ne):
        return idx.reshape(NW, ne // NW // C, C)

    src_gg, dst_gg = chunked(gene_go[0], e_gg), chunked(gene_go[1], e_gg)
    src_gk, dst_gk = chunked(go_ke[0], e_gk), chunked(go_ke[1], e_gk)
    src_kk, dst_kk = chunked(ke_ke[0], e_kk), chunked(ke_ke[1], e_kk)

    col = lambda v: v.reshape(-1, 1)

    gene_t = _tc_node_mlp(xg_t, gene_W1, gene_b1, gene_W2, gene_b2, 1024)
    s1, c1 = _make_edge_mean(NG, NGO, e_gg)(gene_t, src_gg, dst_gg, zeros, ones)
    godec_t = _tc_go_encode(s1, c1, go_enc_W, col(go_enc_b),
                            go_dec_W1, go_dec_b1, go_dec_W2, go_dec_b2)

    s2, c2 = _make_edge_mean(NGO, NKE, e_gk)(godec_t, src_gk, dst_gk, zeros, ones)
    ke0 = _tc_mean(s2, c2)

    s3, c3 = _make_edge_mean(NKE, NKE, e_kk)(ke0, src_kk, dst_kk, zeros, ones)
    ke1 = _tc_ke_mix(s3, c3, ke0, col(ke_ws0), col(ke_wn0), col(ke_b0))
    s4 = _make_edge_mean(NKE, NKE, e_kk, with_counts=False)(
        ke1, src_kk, dst_kk, zeros, ones)
    ke2 = _tc_ke_mix_mlp(s4, c3, ke1, col(ke_ws1), col(ke_wn1), col(ke_b1),
                         kel_W1, kel_b1, kel_W2, kel_b2)

    bio_t = _make_row_gather(NKE, NT)(ke2, tissue)
    out_t = _tc_heads(bio_t, drug_t, bio_W1, col(bio_b1), bio_W2, col(bio_b2),
                      drug_W1, col(drug_b1), drug_W2, col(drug_b2),
                      pred_W, col(pred_b))
    return out_t.reshape(B, 1)


# speedup vs baseline: 1.8959x; 1.5499x over previous
import functools

import jax
import jax.numpy as jnp
from jax import lax
from jax.experimental import pallas as pl
from jax.experimental.pallas import tpu as pltpu
from jax.experimental.pallas import tpu_sc as plsc

B = 256
NG = 8192
NGO = 4096
NKE = 1024
DRUG = 2048
NT = 512
H = 8
NC = 2
NS = 16
NW = NC * NS
C = 128

_F32 = jnp.float32
_HIGH = lax.Precision.HIGHEST


def _dotT(w, x):
    return lax.dot_general(w, x, (((0,), (0,)), ((), ())),
                           preferred_element_type=_F32, precision=_HIGH)


@functools.lru_cache(maxsize=None)
def _make_edge_mean(nsrc, ndst, ne, with_counts=True, stage_spmem=False):
    epw = ne // NW
    nch = epw // C
    rps = ndst // NS
    xps = nsrc // NS
    mesh = plsc.VectorSubcoreMesh(core_axis_name="c", subcore_axis_name="s", num_cores=NC, num_subcores=NS)

    out_type = [jax.ShapeDtypeStruct((NC, ndst, B), _F32)]
    scratch = [
        pltpu.VMEM((nch, C), jnp.int32),
        pltpu.VMEM((nch, C), jnp.int32),
        pltpu.VMEM((C, B), _F32),
        pltpu.VMEM((C, B), _F32),
        pltpu.SemaphoreType.DMA,
        pltpu.SemaphoreType.DMA,
        pltpu.SemaphoreType.DMA,
        pltpu.SemaphoreType.DMA,
    ]
    if with_counts:
        out_type.append(jax.ShapeDtypeStruct((NW, ndst), _F32))
        scratch.append(pltpu.VMEM((ndst,), _F32))
    if stage_spmem:
        scratch.append(pltpu.VMEM_SHARED((nsrc, B), _F32))

    @functools.partial(
        pl.kernel,
        out_type=tuple(out_type) if with_counts else out_type[0],
        mesh=mesh,
        compiler_params=pltpu.CompilerParams(needs_layout_passes=False),
        scratch_types=scratch,
    )
    def ker(x_hbm, src_hbm, dst_hbm, zeros_hbm, sums_hbm, *rest):
        rest = list(rest)
        x_sh = rest.pop() if stage_spmem else None
        if with_counts:
            (cnts_hbm, srcv, dstv, rows0, rows1,
             gsem0, gsem1, ssem0, ssem1, hist) = rest
        else:
            srcv, dstv, rows0, rows1, gsem0, gsem1, ssem0, ssem1 = rest
        cid = lax.axis_index("c")
        sid = lax.axis_index("s")
        w = cid * NS + sid
        pltpu.sync_copy(zeros_hbm.at[pl.ds(0, rps)],
                        sums_hbm.at[cid, pl.ds(sid * rps, rps)])
        if stage_spmem:
            pltpu.sync_copy(x_hbm.at[pl.ds(sid * xps, xps)],
                            x_sh.at[pl.ds(sid * xps, xps)])
        pltpu.sync_copy(src_hbm.at[w], srcv)
        pltpu.sync_copy(dst_hbm.at[w], dstv)
        z16 = jnp.zeros((16,), _F32)
        if with_counts:
            def zb(r, carry):
                hist[pl.ds(r * 16, 16)] = z16
                return carry
            lax.fori_loop(0, ndst // 16, zb, 0)
        plsc.subcore_barrier()

        o16 = jnp.ones((16,), _F32)

        def hist_update(j):
            if with_counts:
                for k in range(C // 16):
                    dv = dstv[j, pl.ds(k * 16, 16)]
                    plsc.addupdate_scatter(hist, [dv], o16)

        x_src = x_sh if stage_spmem else x_hbm

        pltpu.async_copy(x_src.at[srcv.at[0]], rows0, gsem0)
        pltpu.async_copy(x_src.at[srcv.at[1]], rows1, gsem1)

        def pair(i, carry):
            j = i * 2
            pltpu.make_async_copy(x_src.at[srcv.at[j]], rows0, gsem0).wait()
            pltpu.async_copy(rows0, sums_hbm.at[cid].at[dstv.at[j]], ssem0,
                             add=True)
            hist_update(j)
            pltpu.make_async_copy(x_src.at[srcv.at[j + 1]], rows1, gsem1).wait()
            pltpu.async_copy(rows1, sums_hbm.at[cid].at[dstv.at[j + 1]], ssem1,
                             add=True)
            hist_update(j + 1)
            pltpu.make_async_copy(rows0, sums_hbm.at[cid].at[dstv.at[j]],
                                  ssem0).wait()
            pltpu.make_async_copy(rows1, sums_hbm.at[cid].at[dstv.at[j + 1]],
                                  ssem1).wait()

            @pl.when(i + 1 < nch // 2)
            def _():
                pltpu.async_copy(x_src.at[srcv.at[j + 2]], rows0, gsem0)
                pltpu.async_copy(x_src.at[srcv.at[j + 3]], rows1, gsem1)

            return carry

        lax.fori_loop(0, nch // 2, pair, 0)
        if with_counts:
            pltpu.sync_copy(hist, cnts_hbm.at[w])

    return ker


@functools.lru_cache(maxsize=None)
def _make_row_gather(nsrc, nidx):
    ipw = nidx // NW
    mesh = plsc.VectorSubcoreMesh(core_axis_name="c", subcore_axis_name="s", num_cores=NC, num_subcores=NS)

    @functools.partial(
        pl.kernel,
        out_type=jax.ShapeDtypeStruct((nidx, B), _F32),
        mesh=mesh,
        scratch_types=[
            pltpu.VMEM((ipw,), jnp.int32),
            pltpu.VMEM((ipw, B), _F32),
            pltpu.SemaphoreType.DMA,
        ],
    )
    def ker(x_hbm, idx_hbm, out_hbm, idxv, rows, sem):
        w = lax.axis_index("c") * NS + lax.axis_index("s")
        pltpu.sync_copy(idx_hbm.at[pl.ds(w * ipw, ipw)], idxv)
        pltpu.async_copy(x_hbm.at[idxv], rows, sem).wait()
        pltpu.sync_copy(rows, out_hbm.at[pl.ds(w * ipw, ipw)])

    return ker


def _node_mlp_body(x, w1_ref, b1_ref, w2_ref, b2_ref):
    acc = jnp.zeros_like(x)
    for k in range(H):
        acc = acc + jnp.tanh(x * w1_ref[k] + b1_ref[k]) * w2_ref[k]
    return acc + b2_ref[0]


def _tc_node_mlp(x_t, w1, b1, w2, b2, bn):
    n = x_t.shape[0]

    def body(x_ref, w1_ref, b1_ref, w2_ref, b2_ref, o_ref):
        o_ref[...] = _node_mlp_body(x_ref[...], w1_ref, b1_ref, w2_ref, b2_ref)

    smem = pl.BlockSpec(memory_space=pltpu.SMEM)
    return pl.pallas_call(
        body,
        grid=(n // bn,),
        in_specs=[pl.BlockSpec((bn, B), lambda j: (j, 0)), smem, smem, smem, smem],
        out_specs=pl.BlockSpec((bn, B), lambda j: (j, 0)),
        out_shape=jax.ShapeDtypeStruct((n, B), _F32),
    )(x_t, w1, b1, w2, b2)


def _combine_mean(s_ref, c_ref, ones_ref):
    s = s_ref[0] + s_ref[1]
    c = _dotT(c_ref[...], ones_ref[...])
    return s / jnp.maximum(c, 1.0)


def _tc_mean(sums, cnts, ones32):
    ndst = sums.shape[1]

    def body(s_ref, c_ref, o_ref32, o_ref):
        o_ref[...] = _combine_mean(s_ref, c_ref, o_ref32)

    return pl.pallas_call(
        body,
        out_shape=jax.ShapeDtypeStruct((ndst, B), _F32),
    )(sums, cnts, ones32)


def _tc_go_encode(sums, cnts, ones32, enc_w, enc_b2d, dw1, db1, dw2, db2):
    OB = 512

    def body(s_ref, c_ref, o32_ref, w_ref, b_ref, w1_ref, b1_ref, w2_ref,
             b2_ref, o_ref, m_ref):
        @pl.when(pl.program_id(0) == 0)
        def _():
            m_ref[...] = _combine_mean(s_ref, c_ref, o32_ref)

        g = jnp.tanh(_dotT(w_ref[...], m_ref[...]) + b_ref[...])
        o_ref[...] = _node_mlp_body(g, w1_ref, b1_ref, w2_ref, b2_ref)

    smem = pl.BlockSpec(memory_space=pltpu.SMEM)
    return pl.pallas_call(
        body,
        grid=(NGO // OB,),
        in_specs=[
            pl.BlockSpec((NC, NGO, B), lambda j: (0, 0, 0)),
            pl.BlockSpec((NW, NGO), lambda j: (0, 0)),
            pl.BlockSpec((NW, 1), lambda j: (0, 0)),
            pl.BlockSpec((NGO, OB), lambda j: (0, j)),
            pl.BlockSpec((OB, 1), lambda j: (j, 0)),
            smem, smem, smem, smem,
        ],
        out_specs=pl.BlockSpec((OB, B), lambda j: (j, 0)),
        out_shape=jax.ShapeDtypeStruct((NGO, B), _F32),
        scratch_shapes=[pltpu.VMEM((NGO, B), _F32)],
    )(sums, cnts, ones32, enc_w, enc_b2d, dw1, db1, dw2, db2)


def _tc_ke_mix(sums, cnts, ones32, ke, ws2d, wn2d, b2d):
    def body(s_ref, c_ref, o32_ref, k_ref, ws_ref, wn_ref, b_ref, o_ref):
        agg = _combine_mean(s_ref, c_ref, o32_ref)
        o_ref[...] = jax.nn.relu(k_ref[...] * ws_ref[...] + agg * wn_ref[...]
                                 + b_ref[...])

    return pl.pallas_call(
        body,
        out_shape=jax.ShapeDtypeStruct((NKE, B), _F32),
    )(sums, cnts, ones32, ke, ws2d, wn2d, b2d)


def _tc_ke_mix_mlp(sums, cnts, ones32, ke, ws2d, wn2d, b2d, w1, b1, w2, b2):
    def body(s_ref, c_ref, o32_ref, k_ref, ws_ref, wn_ref, b_ref,
             w1_ref, b1_ref, w2_ref, b2_ref, o_ref):
        agg = _combine_mean(s_ref, c_ref, o32_ref)
        mixed = jax.nn.relu(k_ref[...] * ws_ref[...] + agg * wn_ref[...]
                            + b_ref[...])
        o_ref[...] = _node_mlp_body(mixed, w1_ref, b1_ref, w2_ref, b2_ref)

    smem = pl.BlockSpec(memory_space=pltpu.SMEM)
    vmem = pl.BlockSpec()
    return pl.pallas_call(
        body,
        in_specs=[vmem, vmem, vmem, vmem, vmem, vmem, vmem,
                  smem, smem, smem, smem],
        out_shape=jax.ShapeDtypeStruct((NKE, B), _F32),
    )(sums, cnts, ones32, ke, ws2d, wn2d, b2d, w1, b1, w2, b2)


def _tc_heads(bio_t, drug_t, bw1, bb1, bw2, bb2, dw1, db1, dw2, db2, pw, pb):
    def body(bio_ref, drug_ref, bw1_ref, bb1_ref, bw2_ref, bb2_ref,
             dw1_ref, db1_ref, dw2_ref, db2_ref, pw_ref, pb_ref, o_ref):
        h = jax.nn.relu(_dotT(bw1_ref[...], bio_ref[...]) + bb1_ref[...])
        h = jax.nn.relu(_dotT(bw2_ref[...], h) + bb2_ref[...])
        d = jax.nn.relu(_dotT(dw1_ref[...], drug_ref[...]) + db1_ref[...])
        d = jax.nn.relu(_dotT(dw2_ref[...], d) + db2_ref[...])
        comb = jnp.concatenate([h, d], axis=0)
        o_ref[...] = _dotT(pw_ref[...], comb) + pb_ref[...]

    return pl.pallas_call(
        body,
        out_shape=jax.ShapeDtypeStruct((1, B), _F32),
    )(bio_t, drug_t, bw1, bb1, bw2, bb2, dw1, db1, dw2, db2, pw, pb)


def kernel(input_tensor, gene_W1, gene_b1, gene_W2, gene_b2, go_enc_W,
           go_enc_b, go_dec_W1, go_dec_b1, go_dec_W2, go_dec_b2,
           ke_ws0, ke_wn0, ke_b0, ke_ws1, ke_wn1, ke_b1,
           kel_W1, kel_b1, kel_W2, kel_b2,
           bio_W1, bio_b1, bio_W2, bio_b2,
           drug_W1, drug_b1, drug_W2, drug_b2,
           pred_W, pred_b,
           gene_go, go_ke, ke_ke, tissue):
    e_gg = gene_go.shape[1]
    e_gk = go_ke.shape[1]
    e_kk = ke_ke.shape[1]

    input_t = input_tensor.T
    xg_t = input_t[:NG]
    drug_t = input_t[NG:]

    zeros = jnp.zeros((256, B), _F32)
    ones32 = jnp.ones((NW, 1), _F32)

    def chunked(idx, ne):
        return idx.reshape(NW, ne // NW // C, C)

    src_gg, dst_gg = chunked(gene_go[0], e_gg), chunked(gene_go[1], e_gg)
    src_gk, dst_gk = chunked(go_ke[0], e_gk), chunked(go_ke[1], e_gk)
    src_kk, dst_kk = chunked(ke_ke[0], e_kk), chunked(ke_ke[1], e_kk)

    col = lambda v: v.reshape(-1, 1)

    gene_t = _tc_node_mlp(xg_t, gene_W1, gene_b1, gene_W2, gene_b2, 1024)
    s1, c1 = _make_edge_mean(NG, NGO, e_gg)(gene_t, src_gg, dst_gg, zeros)
    godec_t = _tc_go_encode(s1, c1, ones32, go_enc_W, col(go_enc_b),
                            go_dec_W1, go_dec_b1, go_dec_W2, go_dec_b2)

    s2, c2 = _make_edge_mean(NGO, NKE, e_gk)(godec_t, src_gk, dst_gk, zeros)
    ke0 = _tc_mean(s2, c2, ones32)

    s3, c3 = _make_edge_mean(NKE, NKE, e_kk)(ke0, src_kk, dst_kk, zeros)
    ke1 = _tc_ke_mix(s3, c3, ones32, ke0, col(ke_ws0), col(ke_wn0), col(ke_b0))
    s4 = _make_edge_mean(NKE, NKE, e_kk, with_counts=False)(
        ke1, src_kk, dst_kk, zeros)
    ke2 = _tc_ke_mix_mlp(s4, c3, ones32, ke1, col(ke_ws1), col(ke_wn1), col(ke_b1),
                         kel_W1, kel_b1, kel_W2, kel_b2)

    bio_t = _make_row_gather(NKE, NT)(ke2, tissue)
    out_t = _tc_heads(bio_t, drug_t, bio_W1, col(bio_b1), bio_W2, col(bio_b2),
                      drug_W1, col(drug_b1), drug_W2, col(drug_b2),
                      pred_W, col(pred_b))
    return out_t.reshape(B, 1)
